# baseline (speedup 1.0000x reference)
import functools

import jax
import jax.numpy as jnp
from jax import lax
from jax.experimental import pallas as pl
from jax.experimental.pallas import tpu as pltpu
from jax.experimental.pallas import tpu_sc as plsc

E = 20000
C = 1024
P = 32
PPAD = 128
G = PPAD // P
EW = E // G
L = 2048

NC = 2
NS = 16
NW = NC * NS
BPW = L // NW

LBLK = 512
NLB = L // LBLK
RBLK = 2000
NRB = E // RBLK

_NT = (((1,), (1,)), ((), ()))


def _sc_gather_epw(epww, widx):
    mesh = plsc.VectorSubcoreMesh(core_axis_name="c", subcore_axis_name="s")

    @functools.partial(
        pl.kernel,
        mesh=mesh,
        out_type=jax.ShapeDtypeStruct((L, PPAD), jnp.float32),
        scratch_types=[
            pltpu.VMEM((BPW,), jnp.int32),
            pltpu.VMEM((BPW, PPAD), jnp.float32),
            pltpu.SemaphoreType.DMA,
        ],
    )
    def k(epww_hbm, widx_hbm, gepw_hbm, widx_v, bw, sem):
        wid = lax.axis_index("s") * NC + lax.axis_index("c")
        base = wid * BPW
        pltpu.sync_copy(widx_hbm.at[pl.ds(base, BPW)], widx_v)
        pltpu.async_copy(epww_hbm.at[widx_v], bw, sem).wait()
        pltpu.sync_copy(bw, gepw_hbm.at[pl.ds(base, BPW)])

    return k(epww, widx)


def _gred_body(idx_ref, ecw_hbm, adj_hbm, sc_ref, ccw_ref, a_ref,
               be0, be1, ba0, ba1, se0, se1, sa0, sa1):
    ebufs = (be0, be1)
    abufs = (ba0, ba1)
    esems = (se0, se1)
    asems = (sa0, sa1)

    def issue(k):
        s = k % 2
        base = k * LBLK
        eb, ab, es, asm = ebufs[s], abufs[s], esems[s], asems[s]

        def body(j, carry):
            r = idx_ref[base + j]
            pltpu.make_async_copy(
                ecw_hbm.at[pl.ds(r, 1)], eb.at[pl.ds(j, 1)], es).start()
            pltpu.make_async_copy(
                adj_hbm.at[pl.ds(r, 1)], ab.at[pl.ds(j, 1)], asm).start()
            return carry

        lax.fori_loop(0, LBLK, body, 0, unroll=8)

    def drain(k):
        s = k % 2
        pltpu.make_async_copy(
            ecw_hbm.at[pl.ds(0, LBLK)], ebufs[s], esems[s]).wait()
        pltpu.make_async_copy(
            adj_hbm.at[pl.ds(0, LBLK)], abufs[s], asems[s]).wait()

    issue(0)
    sc = sc_ref[...]
    ones = jnp.ones((1, LBLK), jnp.float32)
    accw = jnp.zeros((1, C), jnp.float32)
    accadj = jnp.zeros((1, C), jnp.float32)
    accxw = jnp.zeros((1, C), jnp.float32)
    for k in range(NLB):
        if k + 1 < NLB:
            issue(k + 1)
        drain(k)
        s = k % 2
        adjb = abufs[s][...]
        wb = jax.nn.sigmoid(ebufs[s][...]) * adjb
        accw = accw + jnp.dot(ones, wb)
        accadj = accadj + jnp.dot(ones, adjb)
        accxw = accxw + jnp.dot(sc[:, k * LBLK:(k + 1) * LBLK], wb)

    mask = accadj > 0.0
    a1 = jnp.where(mask, accxw / accw, 0.0)
    ccw = ccw_ref[...]
    mg = jnp.max(ccw, axis=0, keepdims=True)
    ex = jnp.exp(ccw - mg)
    numer = jnp.dot(a1, ex)
    denom = jnp.dot(mask.astype(jnp.float32), ex)
    a_ref[...] = numer / denom


def _tc_gather_reduce(idx, ecw, adj, scores, ccw):
    return pl.pallas_call(
        _gred_body,
        in_specs=[
            pl.BlockSpec(memory_space=pltpu.SMEM),
            pl.BlockSpec(memory_space=pltpu.HBM),
            pl.BlockSpec(memory_space=pltpu.HBM),
            pl.BlockSpec((1, L), lambda: (0, 0)),
            pl.BlockSpec((C, C), lambda: (0, 0)),
        ],
        out_specs=pl.BlockSpec((1, C), lambda: (0, 0)),
        out_shape=jax.ShapeDtypeStruct((1, C), jnp.float32),
        scratch_shapes=[
            pltpu.VMEM((LBLK, C), jnp.float32),
            pltpu.VMEM((LBLK, C), jnp.float32),
            pltpu.VMEM((LBLK, C), jnp.float32),
            pltpu.VMEM((LBLK, C), jnp.float32),
            pltpu.SemaphoreType.DMA,
            pltpu.SemaphoreType.DMA,
            pltpu.SemaphoreType.DMA,
            pltpu.SemaphoreType.DMA,
        ],
    )(idx, ecw, adj, scores, ccw)


def _bm_body(gepw_ref, off_ref, sc_ref, bm_ref):
    wide = gepw_ref[...]
    offc = off_ref[...]
    grp = lax.broadcasted_iota(jnp.int32, (L, PPAD), 1) // P
    sel = grp == offc
    m3w = jnp.max(wide, axis=0, keepdims=True)
    m3 = jnp.maximum(
        jnp.maximum(m3w[:, 0:P], m3w[:, P:2 * P]),
        jnp.maximum(m3w[:, 2 * P:3 * P], m3w[:, 3 * P:4 * P]))
    m3b = jnp.concatenate([m3, m3, m3, m3], axis=1)
    e3 = jnp.where(sel, jnp.exp(wide - m3b), 0.0)
    s3w = jnp.sum(e3, axis=0, keepdims=True)
    t3w = jnp.dot(sc_ref[...], e3)
    s3 = (s3w[:, 0:P] + s3w[:, P:2 * P]
          + s3w[:, 2 * P:3 * P] + s3w[:, 3 * P:4 * P])
    t3 = (t3w[:, 0:P] + t3w[:, P:2 * P]
          + t3w[:, 2 * P:3 * P] + t3w[:, 3 * P:4 * P])
    bm_ref[...] = t3 / s3


def _tc_bm(gepw, off_col, scores):
    return pl.pallas_call(
        _bm_body,
        in_specs=[
            pl.BlockSpec((L, PPAD), lambda: (0, 0)),
            pl.BlockSpec((L, 1), lambda: (0, 0)),
            pl.BlockSpec((1, L), lambda: (0, 0)),
        ],
        out_specs=pl.BlockSpec((1, P), lambda: (0, 0)),
        out_shape=jax.ShapeDtypeStruct((1, P), jnp.float32),
    )(gepw, off_col, scores)


def _stream_body(ecw_ref, adj_ref, epw_ref, lam_ref, gue_ref, sli_ref,
                 a_ref, bm_ref, y_ref):
    adj_blk = adj_ref[...]
    w = jax.nn.sigmoid(ecw_ref[...]) * adj_blk
    ones = jnp.ones((1, C), jnp.float32)
    s = lax.dot_general(ones, w, _NT)
    num = lax.dot_general(a_ref[...], w, _NT)
    ya = num / s
    d = epw_ref[...]
    e3 = jnp.exp(d - jnp.max(d, axis=1, keepdims=True))
    d2n = e3 / jnp.sum(e3, axis=1, keepdims=True)
    yb = lax.dot_general(bm_ref[...], d2n, _NT)
    lam = jax.nn.sigmoid(lam_ref[0])
    gue = jax.nn.sigmoid(gue_ref[0])
    sli = jax.nn.sigmoid(sli_ref[0])
    y_ = (1.0 - lam) * ya + lam * yb
    y_ = jnp.clip(y_, 1e-8, 1.0 - 1e-8)
    y_ref[0] = (1.0 - sli) * y_ + gue * (1.0 - y_)


def _tc_stream(ecw, adj, epw, lam3, gue3, sli3, a, bm):
    return pl.pallas_call(
        _stream_body,
        grid=(NRB,),
        in_specs=[
            pl.BlockSpec((RBLK, C), lambda i: (i, 0)),
            pl.BlockSpec((RBLK, C), lambda i: (i, 0)),
            pl.BlockSpec((RBLK, P), lambda i: (i, 0)),
            pl.BlockSpec((1, 1, RBLK), lambda i: (i, 0, 0)),
            pl.BlockSpec((1, 1, RBLK), lambda i: (i, 0, 0)),
            pl.BlockSpec((1, 1, RBLK), lambda i: (i, 0, 0)),
            pl.BlockSpec((1, C), lambda i: (0, 0)),
            pl.BlockSpec((1, P), lambda i: (0, 0)),
        ],
        out_specs=pl.BlockSpec((1, 1, RBLK), lambda i: (i, 0, 0)),
        out_shape=jax.ShapeDtypeStruct((NRB, 1, RBLK), jnp.float32),
        compiler_params=pltpu.CompilerParams(
            dimension_semantics=("arbitrary",)),
    )(ecw, adj, epw, lam3, gue3, sli3, a, bm)


def kernel(exer_list, score_list, school_feature, exer_conc_adj,
           school_feature_dim_w, exer_conc_w, conc_conc_w, exer_pote_w,
           lambd, guess, slide):
    del school_feature, school_feature_dim_w
    idx = exer_list.reshape(L).astype(jnp.int32)
    widx = idx // G
    off_col = (idx % G).reshape(L, 1)
    scores = score_list.reshape(1, L).astype(jnp.float32)
    epww = exer_pote_w.reshape(EW, PPAD)
    gepw = _sc_gather_epw(epww, widx)
    a = _tc_gather_reduce(idx, exer_conc_w, exer_conc_adj, scores,
                          conc_conc_w)
    bm = _tc_bm(gepw, off_col, scores)
    lam3 = lambd.reshape(NRB, 1, RBLK)
    gue3 = guess.reshape(NRB, 1, RBLK)
    sli3 = slide.reshape(NRB, 1, RBLK)
    y3 = _tc_stream(exer_conc_w, exer_conc_adj, exer_pote_w,
                    lam3, gue3, sli3, a, bm)
    return (a, y3.reshape(1, E))


# speedup vs baseline: 1.0536x; 1.0536x over previous
import functools

import jax
import jax.numpy as jnp
from jax import lax
from jax.experimental import pallas as pl
from jax.experimental.pallas import tpu as pltpu
from jax.experimental.pallas import tpu_sc as plsc

E = 20000
C = 1024
P = 32
PPAD = 128
G = PPAD // P
EW = E // G
L = 2048

NC = 2
NS = 16
NW = NC * NS
BPW = L // NW

LBLK = 512
NLB = L // LBLK
RBLK = 2000
NRB = E // RBLK

_NT = (((1,), (1,)), ((), ()))


def _sc_gather_epw(epww, widx):
    mesh = plsc.VectorSubcoreMesh(core_axis_name="c", subcore_axis_name="s")

    @functools.partial(
        pl.kernel,
        mesh=mesh,
        out_type=jax.ShapeDtypeStruct((L, PPAD), jnp.float32),
        scratch_types=[
            pltpu.VMEM((BPW,), jnp.int32),
            pltpu.VMEM((BPW, PPAD), jnp.float32),
            pltpu.SemaphoreType.DMA,
        ],
    )
    def k(epww_hbm, widx_hbm, gepw_hbm, widx_v, bw, sem):
        wid = lax.axis_index("s") * NC + lax.axis_index("c")
        base = wid * BPW
        pltpu.sync_copy(widx_hbm.at[pl.ds(base, BPW)], widx_v)
        pltpu.async_copy(epww_hbm.at[widx_v], bw, sem).wait()
        pltpu.sync_copy(bw, gepw_hbm.at[pl.ds(base, BPW)])

    return k(epww, widx)


def _gred_body(idx_ref, ecw_hbm, adj_hbm, epww_hbm, sc_ref, off_ref,
               ccw_ref, a_ref, bm_ref,
               be0, be1, ba0, ba1, bw, se0, se1, sa0, sa1, sw):
    ebufs = (be0, be1)
    abufs = (ba0, ba1)
    esems = (se0, se1)
    asems = (sa0, sa1)

    def issue(k):
        s = k % 2
        base = k * LBLK
        eb, ab, es, asm = ebufs[s], abufs[s], esems[s], asems[s]

        def body(j, carry):
            r = idx_ref[base + j]
            pltpu.make_async_copy(
                ecw_hbm.at[pl.ds(r, 1)], eb.at[pl.ds(j, 1)], es).start()
            pltpu.make_async_copy(
                adj_hbm.at[pl.ds(r, 1)], ab.at[pl.ds(j, 1)], asm).start()
            pltpu.make_async_copy(
                epww_hbm.at[pl.ds(r // G, 1)], bw.at[pl.ds(base + j, 1)],
                sw).start()
            return carry

        lax.fori_loop(0, LBLK, body, 0, unroll=8)

    def drain(k):
        s = k % 2
        pltpu.make_async_copy(
            ecw_hbm.at[pl.ds(0, LBLK)], ebufs[s], esems[s]).wait()
        pltpu.make_async_copy(
            adj_hbm.at[pl.ds(0, LBLK)], abufs[s], asems[s]).wait()

    issue(0)
    sc = sc_ref[...]
    ones = jnp.ones((1, LBLK), jnp.float32)
    accw = jnp.zeros((1, C), jnp.float32)
    accadj = jnp.zeros((1, C), jnp.float32)
    accxw = jnp.zeros((1, C), jnp.float32)
    for k in range(NLB):
        if k + 1 < NLB:
            issue(k + 1)
        drain(k)
        s = k % 2
        adjb = abufs[s][...]
        wb = jax.nn.sigmoid(ebufs[s][...]) * adjb
        accw = accw + jnp.dot(ones, wb)
        accadj = accadj + jnp.dot(ones, adjb)
        accxw = accxw + jnp.dot(sc[:, k * LBLK:(k + 1) * LBLK], wb)

    mask = accadj > 0.0
    a1 = jnp.where(mask, accxw / accw, 0.0)
    ccw = ccw_ref[...]
    mg = jnp.max(ccw, axis=0, keepdims=True)
    ex = jnp.exp(ccw - mg)
    numer = jnp.dot(a1, ex)
    denom = jnp.dot(mask.astype(jnp.float32), ex)
    a_ref[...] = numer / denom

    pltpu.make_async_copy(epww_hbm.at[pl.ds(0, L)], bw, sw).wait()
    wide = bw[...]
    offc = off_ref[...]
    grp = lax.broadcasted_iota(jnp.int32, (L, PPAD), 1) // P
    sel = grp == offc
    m3w = jnp.max(wide, axis=0, keepdims=True)
    m3 = jnp.maximum(
        jnp.maximum(m3w[:, 0:P], m3w[:, P:2 * P]),
        jnp.maximum(m3w[:, 2 * P:3 * P], m3w[:, 3 * P:4 * P]))
    m3b = jnp.concatenate([m3, m3, m3, m3], axis=1)
    e3 = jnp.where(sel, jnp.exp(wide - m3b), 0.0)
    s3w = jnp.sum(e3, axis=0, keepdims=True)
    t3w = jnp.dot(sc, e3)
    s3 = (s3w[:, 0:P] + s3w[:, P:2 * P]
          + s3w[:, 2 * P:3 * P] + s3w[:, 3 * P:4 * P])
    t3 = (t3w[:, 0:P] + t3w[:, P:2 * P]
          + t3w[:, 2 * P:3 * P] + t3w[:, 3 * P:4 * P])
    bm_ref[...] = t3 / s3


def _tc_gather_reduce(idx, ecw, adj, epww, scores, off_col, ccw):
    return pl.pallas_call(
        _gred_body,
        in_specs=[
            pl.BlockSpec(memory_space=pltpu.SMEM),
            pl.BlockSpec(memory_space=pltpu.HBM),
            pl.BlockSpec(memory_space=pltpu.HBM),
            pl.BlockSpec(memory_space=pltpu.HBM),
            pl.BlockSpec((1, L), lambda: (0, 0)),
            pl.BlockSpec((L, 1), lambda: (0, 0)),
            pl.BlockSpec((C, C), lambda: (0, 0)),
        ],
        out_specs=[
            pl.BlockSpec((1, C), lambda: (0, 0)),
            pl.BlockSpec((1, P), lambda: (0, 0)),
        ],
        out_shape=[
            jax.ShapeDtypeStruct((1, C), jnp.float32),
            jax.ShapeDtypeStruct((1, P), jnp.float32),
        ],
        scratch_shapes=[
            pltpu.VMEM((LBLK, C), jnp.float32),
            pltpu.VMEM((LBLK, C), jnp.float32),
            pltpu.VMEM((LBLK, C), jnp.float32),
            pltpu.VMEM((LBLK, C), jnp.float32),
            pltpu.VMEM((L, PPAD), jnp.float32),
            pltpu.SemaphoreType.DMA,
            pltpu.SemaphoreType.DMA,
            pltpu.SemaphoreType.DMA,
            pltpu.SemaphoreType.DMA,
            pltpu.SemaphoreType.DMA,
        ],
    )(idx, ecw, adj, epww, scores, off_col, ccw)


def _stream_body(ecw_ref, adj_ref, epw_ref, lam_ref, gue_ref, sli_ref,
                 a_ref, bm_ref, y_ref):
    adj_blk = adj_ref[...]
    w = jax.nn.sigmoid(ecw_ref[...]) * adj_blk
    ones = jnp.ones((1, C), jnp.float32)
    s = lax.dot_general(ones, w, _NT)
    num = lax.dot_general(a_ref[...], w, _NT)
    ya = num / s
    d = epw_ref[...]
    e3 = jnp.exp(d - jnp.max(d, axis=1, keepdims=True))
    d2n = e3 / jnp.sum(e3, axis=1, keepdims=True)
    yb = lax.dot_general(bm_ref[...], d2n, _NT)
    lam = jax.nn.sigmoid(lam_ref[0])
    gue = jax.nn.sigmoid(gue_ref[0])
    sli = jax.nn.sigmoid(sli_ref[0])
    y_ = (1.0 - lam) * ya + lam * yb
    y_ = jnp.clip(y_, 1e-8, 1.0 - 1e-8)
    y_ref[0] = (1.0 - sli) * y_ + gue * (1.0 - y_)


def _tc_stream(ecw, adj, epw, lam3, gue3, sli3, a, bm):
    return pl.pallas_call(
        _stream_body,
        grid=(NRB,),
        in_specs=[
            pl.BlockSpec((RBLK, C), lambda i: (i, 0)),
            pl.BlockSpec((RBLK, C), lambda i: (i, 0)),
            pl.BlockSpec((RBLK, P), lambda i: (i, 0)),
            pl.BlockSpec((1, 1, RBLK), lambda i: (i, 0, 0)),
            pl.BlockSpec((1, 1, RBLK), lambda i: (i, 0, 0)),
            pl.BlockSpec((1, 1, RBLK), lambda i: (i, 0, 0)),
            pl.BlockSpec((1, C), lambda i: (0, 0)),
            pl.BlockSpec((1, P), lambda i: (0, 0)),
        ],
        out_specs=pl.BlockSpec((1, 1, RBLK), lambda i: (i, 0, 0)),
        out_shape=jax.ShapeDtypeStruct((NRB, 1, RBLK), jnp.float32),
        compiler_params=pltpu.CompilerParams(
            dimension_semantics=("arbitrary",)),
    )(ecw, adj, epw, lam3, gue3, sli3, a, bm)


def kernel(exer_list, score_list, school_feature, exer_conc_adj,
           school_feature_dim_w, exer_conc_w, conc_conc_w, exer_pote_w,
           lambd, guess, slide):
    del school_feature, school_feature_dim_w
    idx = exer_list.reshape(L).astype(jnp.int32)
    widx = idx // G
    off_col = (idx % G).reshape(L, 1)
    scores = score_list.reshape(1, L).astype(jnp.float32)
    epww = exer_pote_w.reshape(EW, PPAD)
    del widx
    a, bm = _tc_gather_reduce(idx, exer_conc_w, exer_conc_adj, epww,
                              scores, off_col, conc_conc_w)
    lam3 = lambd.reshape(NRB, 1, RBLK)
    gue3 = guess.reshape(NRB, 1, RBLK)
    sli3 = slide.reshape(NRB, 1, RBLK)
    y3 = _tc_stream(exer_conc_w, exer_conc_adj, exer_pote_w,
                    lam3, gue3, sli3, a, bm)
    return (a, y3.reshape(1, E))
